# Initial kernel scaffold; baseline (speedup 1.0000x reference)
import jax
import jax.numpy as jnp
from jax.experimental import pallas as pl


def kernel(x, edge_index, W1_l, b1, W1_r, W2_l, b2, W2_r):
    raise NotImplementedError("write your pallas kernel here")


# speedup vs baseline: 5.8636x; 5.8636x over previous
import functools

import jax
import jax.numpy as jnp
from jax import lax
from jax.experimental import pallas as pl
from jax.experimental.pallas import tpu as pltpu
from jax.experimental.pallas import tpu_sc as plsc

_N = 10000
_E = 320000
_D = 128
_O = 64

_NC = 2
_NS = 16
_CH = 80
_EPT = _E // _NS
_NCHUNK = _EPT // _CH
_NPAD = 10112
_RPT = _NPAD // _NS
_DEGW = 8


def _sc_agg_body(with_deg, *refs):
    if with_deg:
        (y_hbm, src_hbm, dst_hbm, zrow_hbm, zdeg_hbm, ones_hbm, agg_out,
         deg_out, srcv, dstv, rows, onesv, agg_s, deg_s, sem) = refs
    else:
        (y_hbm, src_hbm, dst_hbm, zrow_hbm, agg_out,
         srcv, dstv, rows, agg_s, sem) = refs

    cid = lax.axis_index("c")
    sid = lax.axis_index("s")

    r0 = sid * _RPT
    pltpu.sync_copy(zrow_hbm.at[pl.ds(r0, _RPT)], agg_s.at[pl.ds(r0, _RPT)])
    if with_deg:
        pltpu.sync_copy(zdeg_hbm.at[pl.ds(r0, _RPT)], deg_s.at[pl.ds(r0, _RPT)])
        pltpu.sync_copy(ones_hbm, onesv)

    pltpu.sync_copy(src_hbm.at[sid], srcv)
    pltpu.sync_copy(dst_hbm.at[sid], dstv)
    plsc.subcore_barrier()

    def chunk(c, _):
        pltpu.async_copy(y_hbm.at[cid].at[srcv.at[c]], rows, sem).wait()
        pltpu.sync_copy(rows, agg_s.at[dstv.at[c]], add=True)
        if with_deg:
            @pl.when(cid == 0)
            def _():
                pltpu.sync_copy(onesv, deg_s.at[dstv.at[c]], add=True)
        return 0

    lax.fori_loop(0, _NCHUNK, chunk, 0)

    plsc.subcore_barrier()
    pltpu.sync_copy(agg_s.at[pl.ds(r0, _RPT)], agg_out.at[cid, pl.ds(r0, _RPT)])
    if with_deg:
        @pl.when(cid == 0)
        def _():
            pltpu.sync_copy(deg_s.at[pl.ds(r0, _RPT)], deg_out.at[pl.ds(r0, _RPT)])


def _make_sc_agg(d, with_deg):
    mesh = plsc.VectorSubcoreMesh(core_axis_name="c", subcore_axis_name="s")
    out_type = [jax.ShapeDtypeStruct((_NC, _NPAD, d), jnp.float32)]
    scratch = [
        pltpu.VMEM((_NCHUNK, _CH), jnp.int32),
        pltpu.VMEM((_NCHUNK, _CH), jnp.int32),
        pltpu.VMEM((_CH, d), jnp.float32),
    ]
    if with_deg:
        out_type.append(jax.ShapeDtypeStruct((_NPAD, _DEGW), jnp.float32))
        scratch.append(pltpu.VMEM((_CH, _DEGW), jnp.float32))
        scratch.append(pltpu.VMEM_SHARED((_NPAD, d), jnp.float32))
        scratch.append(pltpu.VMEM_SHARED((_NPAD, _DEGW), jnp.float32))
    else:
        scratch.append(pltpu.VMEM_SHARED((_NPAD, d), jnp.float32))
    scratch.append(pltpu.SemaphoreType.DMA)
    return pl.kernel(
        functools.partial(_sc_agg_body, with_deg),
        out_type=out_type,
        mesh=mesh,
        scratch_types=scratch,
        compiler_params=pltpu.CompilerParams(use_tc_tiling_on_sc=False),
    )


_sc_agg_deg = _make_sc_agg(_D // 2, True)
_sc_agg_only = _make_sc_agg(_O // 2, False)


def _tc1_body(agg_ref, deg_ref, x_ref, w1l_ref, b1_ref, w1r_ref,
              w2l_ref, w2r_ref, y2_ref, r2_ref):
    agg = jnp.concatenate([agg_ref[0], agg_ref[1]], axis=1)
    deg = deg_ref[:, 0:1]
    mean = agg * (1.0 / jnp.maximum(deg, 1.0))
    h = mean @ w1l_ref[...] + b1_ref[...] + x_ref[...] @ w1r_ref[...]
    h = jnp.maximum(h, 0.0)
    y2_ref[0] = h @ w2l_ref[:, : _O // 2]
    y2_ref[1] = h @ w2l_ref[:, _O // 2 :]
    r2_ref[...] = h @ w2r_ref[...]


def _tc2_body(agg_ref, deg_ref, r2_ref, b2_ref, z_ref):
    agg = jnp.concatenate([agg_ref[0], agg_ref[1]], axis=1)
    deg = deg_ref[:, 0:1]
    z_ref[...] = agg * (1.0 / jnp.maximum(deg, 1.0)) + b2_ref[...] + r2_ref[...]


_TC_R = 1000


def _tc1(agg, degp, x, w1l, b1, w1r, w2l, w2r):
    nb = _N // _TC_R
    return pl.pallas_call(
        _tc1_body,
        grid=(nb,),
        in_specs=[
            pl.BlockSpec((_NC, _TC_R, _D // 2), lambda m: (0, m, 0)),
            pl.BlockSpec((_TC_R, _DEGW), lambda m: (m, 0)),
            pl.BlockSpec((_TC_R, _D), lambda m: (m, 0)),
            pl.BlockSpec((_D, _D), lambda m: (0, 0)),
            pl.BlockSpec((1, _D), lambda m: (0, 0)),
            pl.BlockSpec((_D, _D), lambda m: (0, 0)),
            pl.BlockSpec((_D, _O), lambda m: (0, 0)),
            pl.BlockSpec((_D, _O), lambda m: (0, 0)),
        ],
        out_specs=[
            pl.BlockSpec((_NC, _TC_R, _O // 2), lambda m: (0, m, 0)),
            pl.BlockSpec((_TC_R, _O), lambda m: (m, 0)),
        ],
        out_shape=[
            jax.ShapeDtypeStruct((_NC, _N, _O // 2), jnp.float32),
            jax.ShapeDtypeStruct((_N, _O), jnp.float32),
        ],
    )(agg, degp, x, w1l, b1, w1r, w2l, w2r)


def _tc2(agg2, degp, r2, b2):
    nb = _N // _TC_R
    return pl.pallas_call(
        _tc2_body,
        grid=(nb,),
        in_specs=[
            pl.BlockSpec((_NC, _TC_R, _O // 2), lambda m: (0, m, 0)),
            pl.BlockSpec((_TC_R, _DEGW), lambda m: (m, 0)),
            pl.BlockSpec((_TC_R, _O), lambda m: (m, 0)),
            pl.BlockSpec((1, _O), lambda m: (0, 0)),
        ],
        out_specs=pl.BlockSpec((_TC_R, _O), lambda m: (m, 0)),
        out_shape=jax.ShapeDtypeStruct((_N, _O), jnp.float32),
    )(agg2, degp, r2, b2)


def kernel(x, edge_index, W1_l, b1, W1_r, W2_l, b2, W2_r):
    src = edge_index[0].reshape(_NS, _NCHUNK, _CH)
    dst = edge_index[1].reshape(_NS, _NCHUNK, _CH)
    x_split = jnp.stack([x[:, : _D // 2], x[:, _D // 2 :]])
    zrow = jnp.zeros((_NPAD, _D // 2), jnp.float32)
    zdeg = jnp.zeros((_NPAD, _DEGW), jnp.float32)
    zout = jnp.zeros((_NPAD, _O // 2), jnp.float32)
    ones = jnp.ones((_CH, _DEGW), jnp.float32)

    agg1, degp = _sc_agg_deg(x_split, src, dst, zrow, zdeg, ones)
    y2s, r2 = _tc1(agg1[:, : _N], degp[: _N], x, W1_l, b1.reshape(1, _D),
                   W1_r, W2_l, W2_r)
    (agg2,) = _sc_agg_only(y2s, src, dst, zout)
    return _tc2(agg2[:, : _N], degp[: _N], r2, b2.reshape(1, _O))


# speedup vs baseline: 9.3276x; 1.5908x over previous
import functools

import jax
import jax.numpy as jnp
from jax import lax
from jax.experimental import pallas as pl
from jax.experimental.pallas import tpu as pltpu
from jax.experimental.pallas import tpu_sc as plsc

_N = 10000
_E = 320000
_D = 128
_O = 64

_NC = 2
_NS = 16
_CH = 80
_EPT = _E // _NS
_NCHUNK = _EPT // _CH
_NPAD = 10112
_RPT = _NPAD // _NS
_DEGW = 8


def _sc_agg_body(with_deg, *refs):
    if with_deg:
        (y_hbm, src_hbm, dst_hbm, zrow_hbm, zdeg_hbm, ones_hbm, agg_out,
         deg_out, srcv, dstv, rows, onesv, agg_s, deg_s, sem0, sem1) = refs
    else:
        (y_hbm, src_hbm, dst_hbm, zrow_hbm, agg_out,
         srcv, dstv, rows, agg_s, sem0, sem1) = refs
    sems = (sem0, sem1)

    cid = lax.axis_index("c")
    sid = lax.axis_index("s")

    r0 = sid * _RPT
    pltpu.sync_copy(zrow_hbm.at[pl.ds(r0, _RPT)], agg_s.at[pl.ds(r0, _RPT)])
    if with_deg:
        pltpu.sync_copy(zdeg_hbm.at[pl.ds(r0, _RPT)], deg_s.at[pl.ds(r0, _RPT)])
        pltpu.sync_copy(ones_hbm, onesv)

    pltpu.sync_copy(src_hbm.at[sid], srcv)
    pltpu.sync_copy(dst_hbm.at[sid], dstv)
    plsc.subcore_barrier()

    pltpu.async_copy(y_hbm.at[cid].at[srcv.at[0]], rows.at[0], sems[0])
    pltpu.async_copy(y_hbm.at[cid].at[srcv.at[1]], rows.at[1], sems[1])

    def pair(i, _):
        for b in range(2):
            c = 2 * i + b
            pltpu.make_async_copy(
                y_hbm.at[cid].at[srcv.at[c]], rows.at[b], sems[b]).wait()
            pltpu.sync_copy(rows.at[b], agg_s.at[dstv.at[c]], add=True)
            if with_deg:
                @pl.when(cid == 0)
                def _():
                    pltpu.sync_copy(onesv, deg_s.at[dstv.at[c]], add=True)

            @pl.when(c + 2 < _NCHUNK)
            def _():
                pltpu.async_copy(
                    y_hbm.at[cid].at[srcv.at[c + 2]], rows.at[b], sems[b])
        return 0

    lax.fori_loop(0, _NCHUNK // 2, pair, 0)

    plsc.subcore_barrier()
    pltpu.sync_copy(agg_s.at[pl.ds(r0, _RPT)], agg_out.at[cid, pl.ds(r0, _RPT)])
    if with_deg:
        @pl.when(cid == 0)
        def _():
            pltpu.sync_copy(deg_s.at[pl.ds(r0, _RPT)], deg_out.at[pl.ds(r0, _RPT)])


def _make_sc_agg(d, with_deg):
    mesh = plsc.VectorSubcoreMesh(core_axis_name="c", subcore_axis_name="s")
    out_type = [jax.ShapeDtypeStruct((_NC, _NPAD, d), jnp.float32)]
    scratch = [
        pltpu.VMEM((_NCHUNK, _CH), jnp.int32),
        pltpu.VMEM((_NCHUNK, _CH), jnp.int32),
        pltpu.VMEM((2, _CH, d), jnp.float32),
    ]
    if with_deg:
        out_type.append(jax.ShapeDtypeStruct((_NPAD, _DEGW), jnp.float32))
        scratch.append(pltpu.VMEM((_CH, _DEGW), jnp.float32))
        scratch.append(pltpu.VMEM_SHARED((_NPAD, d), jnp.float32))
        scratch.append(pltpu.VMEM_SHARED((_NPAD, _DEGW), jnp.float32))
    else:
        scratch.append(pltpu.VMEM_SHARED((_NPAD, d), jnp.float32))
    scratch.append(pltpu.SemaphoreType.DMA)
    scratch.append(pltpu.SemaphoreType.DMA)
    return pl.kernel(
        functools.partial(_sc_agg_body, with_deg),
        out_type=out_type,
        mesh=mesh,
        scratch_types=scratch,
        compiler_params=pltpu.CompilerParams(use_tc_tiling_on_sc=False),
    )


_sc_agg_deg = _make_sc_agg(_D // 2, True)
_sc_agg_only = _make_sc_agg(_O // 2, False)


def _tc1_body(agg_ref, deg_ref, x_ref, w1l_ref, b1_ref, w1r_ref,
              w2l_ref, w2r_ref, y2_ref, r2_ref):
    agg = jnp.concatenate([agg_ref[0], agg_ref[1]], axis=1)
    deg = deg_ref[:, 0:1]
    mean = agg * (1.0 / jnp.maximum(deg, 1.0))
    h = mean @ w1l_ref[...] + b1_ref[...] + x_ref[...] @ w1r_ref[...]
    h = jnp.maximum(h, 0.0)
    y2_ref[0] = h @ w2l_ref[:, : _O // 2]
    y2_ref[1] = h @ w2l_ref[:, _O // 2 :]
    r2_ref[...] = h @ w2r_ref[...]


def _tc2_body(agg_ref, deg_ref, r2_ref, b2_ref, z_ref):
    agg = jnp.concatenate([agg_ref[0], agg_ref[1]], axis=1)
    deg = deg_ref[:, 0:1]
    z_ref[...] = agg * (1.0 / jnp.maximum(deg, 1.0)) + b2_ref[...] + r2_ref[...]


_TC_R = 1000


def _tc1(agg, degp, x, w1l, b1, w1r, w2l, w2r):
    nb = _N // _TC_R
    return pl.pallas_call(
        _tc1_body,
        grid=(nb,),
        in_specs=[
            pl.BlockSpec((_NC, _TC_R, _D // 2), lambda m: (0, m, 0)),
            pl.BlockSpec((_TC_R, _DEGW), lambda m: (m, 0)),
            pl.BlockSpec((_TC_R, _D), lambda m: (m, 0)),
            pl.BlockSpec((_D, _D), lambda m: (0, 0)),
            pl.BlockSpec((1, _D), lambda m: (0, 0)),
            pl.BlockSpec((_D, _D), lambda m: (0, 0)),
            pl.BlockSpec((_D, _O), lambda m: (0, 0)),
            pl.BlockSpec((_D, _O), lambda m: (0, 0)),
        ],
        out_specs=[
            pl.BlockSpec((_NC, _TC_R, _O // 2), lambda m: (0, m, 0)),
            pl.BlockSpec((_TC_R, _O), lambda m: (m, 0)),
        ],
        out_shape=[
            jax.ShapeDtypeStruct((_NC, _N, _O // 2), jnp.float32),
            jax.ShapeDtypeStruct((_N, _O), jnp.float32),
        ],
    )(agg, degp, x, w1l, b1, w1r, w2l, w2r)


def _tc2(agg2, degp, r2, b2):
    nb = _N // _TC_R
    return pl.pallas_call(
        _tc2_body,
        grid=(nb,),
        in_specs=[
            pl.BlockSpec((_NC, _TC_R, _O // 2), lambda m: (0, m, 0)),
            pl.BlockSpec((_TC_R, _DEGW), lambda m: (m, 0)),
            pl.BlockSpec((_TC_R, _O), lambda m: (m, 0)),
            pl.BlockSpec((1, _O), lambda m: (0, 0)),
        ],
        out_specs=pl.BlockSpec((_TC_R, _O), lambda m: (m, 0)),
        out_shape=jax.ShapeDtypeStruct((_N, _O), jnp.float32),
    )(agg2, degp, r2, b2)


def kernel(x, edge_index, W1_l, b1, W1_r, W2_l, b2, W2_r):
    src = edge_index[0].reshape(_NS, _NCHUNK, _CH)
    dst = edge_index[1].reshape(_NS, _NCHUNK, _CH)
    x_split = jnp.stack([x[:, : _D // 2], x[:, _D // 2 :]])
    zrow = jnp.zeros((_NPAD, _D // 2), jnp.float32)
    zdeg = jnp.zeros((_NPAD, _DEGW), jnp.float32)
    zout = jnp.zeros((_NPAD, _O // 2), jnp.float32)
    ones = jnp.ones((_CH, _DEGW), jnp.float32)

    agg1, degp = _sc_agg_deg(x_split, src, dst, zrow, zdeg, ones)
    y2s, r2 = _tc1(agg1[:, : _N], degp[: _N], x, W1_l, b1.reshape(1, _D),
                   W1_r, W2_l, W2_r)
    (agg2,) = _sc_agg_only(y2s, src, dst, zout)
    return _tc2(agg2[:, : _N], degp[: _N], r2, b2.reshape(1, _O))


# speedup vs baseline: 10.4073x; 1.1158x over previous
import functools

import jax
import jax.numpy as jnp
from jax import lax
from jax.experimental import pallas as pl
from jax.experimental.pallas import tpu as pltpu
from jax.experimental.pallas import tpu_sc as plsc

_N = 10000
_E = 320000
_D = 128
_O = 64

_NC = 2
_NS = 16
_CH = 128
_EPT = _E // _NS
_NMAIN = _EPT // _CH
_TAIL = _EPT - _NMAIN * _CH
_NPAD = 10112
_RPT = _NPAD // _NS
_DEGW = 8


def _sc_agg_body(with_deg, *refs):
    if with_deg:
        (y_hbm, src_hbm, dst_hbm, srct_hbm, dstt_hbm, zrow_hbm, zdeg_hbm,
         ones_hbm, agg_out, deg_out, srcv, dstv, srctv, dsttv, rows, rowst,
         onesv, agg_s, deg_s, sem0, sem1) = refs
    else:
        (y_hbm, src_hbm, dst_hbm, srct_hbm, dstt_hbm, zrow_hbm, agg_out,
         srcv, dstv, srctv, dsttv, rows, rowst, agg_s, sem0, sem1) = refs
    sems = (sem0, sem1)

    cid = lax.axis_index("c")
    sid = lax.axis_index("s")

    r0 = sid * _RPT
    pltpu.sync_copy(zrow_hbm.at[pl.ds(r0, _RPT)], agg_s.at[pl.ds(r0, _RPT)])
    if with_deg:
        pltpu.sync_copy(zdeg_hbm.at[pl.ds(r0, _RPT)], deg_s.at[pl.ds(r0, _RPT)])
        pltpu.sync_copy(ones_hbm, onesv)

    pltpu.sync_copy(src_hbm.at[sid], srcv)
    pltpu.sync_copy(dst_hbm.at[sid], dstv)
    pltpu.sync_copy(srct_hbm.at[sid], srctv)
    pltpu.sync_copy(dstt_hbm.at[sid], dsttv)
    plsc.subcore_barrier()

    pltpu.async_copy(y_hbm.at[cid].at[srcv.at[0]], rows.at[0], sems[0])
    pltpu.async_copy(y_hbm.at[cid].at[srcv.at[1]], rows.at[1], sems[1])

    def pair(i, _):
        for b in range(2):
            c = 2 * i + b
            pltpu.make_async_copy(
                y_hbm.at[cid].at[srcv.at[c]], rows.at[b], sems[b]).wait()
            pltpu.sync_copy(rows.at[b], agg_s.at[dstv.at[c]], add=True)
            if with_deg:
                @pl.when(cid == 0)
                def _():
                    pltpu.sync_copy(onesv, deg_s.at[dstv.at[c]], add=True)

            @pl.when(c + 2 < _NMAIN)
            def _():
                pltpu.async_copy(
                    y_hbm.at[cid].at[srcv.at[c + 2]], rows.at[b], sems[b])
        return 0

    lax.fori_loop(0, _NMAIN // 2, pair, 0)

    pltpu.async_copy(y_hbm.at[cid].at[srctv.at[0]], rowst, sems[0]).wait()
    pltpu.sync_copy(rowst, agg_s.at[dsttv.at[0]], add=True)
    if with_deg:
        @pl.when(cid == 0)
        def _():
            pltpu.sync_copy(onesv.at[pl.ds(0, _TAIL)],
                            deg_s.at[dsttv.at[0]], add=True)

    plsc.subcore_barrier()
    pltpu.sync_copy(agg_s.at[pl.ds(r0, _RPT)], agg_out.at[cid, pl.ds(r0, _RPT)])
    if with_deg:
        @pl.when(cid == 0)
        def _():
            pltpu.sync_copy(deg_s.at[pl.ds(r0, _RPT)], deg_out.at[pl.ds(r0, _RPT)])


def _make_sc_agg(d, with_deg):
    mesh = plsc.VectorSubcoreMesh(core_axis_name="c", subcore_axis_name="s")
    out_type = [jax.ShapeDtypeStruct((_NC, _NPAD, d), jnp.float32)]
    scratch = [
        pltpu.VMEM((_NMAIN, _CH), jnp.int32),
        pltpu.VMEM((_NMAIN, _CH), jnp.int32),
        pltpu.VMEM((1, _TAIL), jnp.int32),
        pltpu.VMEM((1, _TAIL), jnp.int32),
        pltpu.VMEM((2, _CH, d), jnp.float32),
        pltpu.VMEM((_TAIL, d), jnp.float32),
    ]
    if with_deg:
        out_type.append(jax.ShapeDtypeStruct((_NPAD, _DEGW), jnp.float32))
        scratch.append(pltpu.VMEM((_CH, _DEGW), jnp.float32))
        scratch.append(pltpu.VMEM_SHARED((_NPAD, d), jnp.float32))
        scratch.append(pltpu.VMEM_SHARED((_NPAD, _DEGW), jnp.float32))
    else:
        scratch.append(pltpu.VMEM_SHARED((_NPAD, d), jnp.float32))
    scratch.append(pltpu.SemaphoreType.DMA)
    scratch.append(pltpu.SemaphoreType.DMA)
    return pl.kernel(
        functools.partial(_sc_agg_body, with_deg),
        out_type=out_type,
        mesh=mesh,
        scratch_types=scratch,
        compiler_params=pltpu.CompilerParams(use_tc_tiling_on_sc=False),
    )


_sc_agg_deg = _make_sc_agg(_D // 2, True)
_sc_agg_only = _make_sc_agg(_O // 2, False)


def _tc1_body(agg_ref, deg_ref, x_ref, w1l_ref, b1_ref, w1r_ref,
              w2l_ref, w2r_ref, y2_ref, r2_ref):
    agg = jnp.concatenate([agg_ref[0], agg_ref[1]], axis=1)
    deg = deg_ref[:, 0:1]
    mean = agg * (1.0 / jnp.maximum(deg, 1.0))
    h = mean @ w1l_ref[...] + b1_ref[...] + x_ref[...] @ w1r_ref[...]
    h = jnp.maximum(h, 0.0)
    y2_ref[0] = h @ w2l_ref[:, : _O // 2]
    y2_ref[1] = h @ w2l_ref[:, _O // 2 :]
    r2_ref[...] = h @ w2r_ref[...]


def _tc2_body(agg_ref, deg_ref, r2_ref, b2_ref, z_ref):
    agg = jnp.concatenate([agg_ref[0], agg_ref[1]], axis=1)
    deg = deg_ref[:, 0:1]
    z_ref[...] = agg * (1.0 / jnp.maximum(deg, 1.0)) + b2_ref[...] + r2_ref[...]


_TC_R = 1000


def _tc1(agg, degp, x, w1l, b1, w1r, w2l, w2r):
    nb = _N // _TC_R
    return pl.pallas_call(
        _tc1_body,
        grid=(nb,),
        in_specs=[
            pl.BlockSpec((_NC, _TC_R, _D // 2), lambda m: (0, m, 0)),
            pl.BlockSpec((_TC_R, _DEGW), lambda m: (m, 0)),
            pl.BlockSpec((_TC_R, _D), lambda m: (m, 0)),
            pl.BlockSpec((_D, _D), lambda m: (0, 0)),
            pl.BlockSpec((1, _D), lambda m: (0, 0)),
            pl.BlockSpec((_D, _D), lambda m: (0, 0)),
            pl.BlockSpec((_D, _O), lambda m: (0, 0)),
            pl.BlockSpec((_D, _O), lambda m: (0, 0)),
        ],
        out_specs=[
            pl.BlockSpec((_NC, _TC_R, _O // 2), lambda m: (0, m, 0)),
            pl.BlockSpec((_TC_R, _O), lambda m: (m, 0)),
        ],
        out_shape=[
            jax.ShapeDtypeStruct((_NC, _N, _O // 2), jnp.float32),
            jax.ShapeDtypeStruct((_N, _O), jnp.float32),
        ],
    )(agg, degp, x, w1l, b1, w1r, w2l, w2r)


def _tc2(agg2, degp, r2, b2):
    nb = _N // _TC_R
    return pl.pallas_call(
        _tc2_body,
        grid=(nb,),
        in_specs=[
            pl.BlockSpec((_NC, _TC_R, _O // 2), lambda m: (0, m, 0)),
            pl.BlockSpec((_TC_R, _DEGW), lambda m: (m, 0)),
            pl.BlockSpec((_TC_R, _O), lambda m: (m, 0)),
            pl.BlockSpec((1, _O), lambda m: (0, 0)),
        ],
        out_specs=pl.BlockSpec((_TC_R, _O), lambda m: (m, 0)),
        out_shape=jax.ShapeDtypeStruct((_N, _O), jnp.float32),
    )(agg2, degp, r2, b2)


def kernel(x, edge_index, W1_l, b1, W1_r, W2_l, b2, W2_r):
    src_f = edge_index[0].reshape(_NS, _EPT)
    dst_f = edge_index[1].reshape(_NS, _EPT)
    src = src_f[:, : _NMAIN * _CH].reshape(_NS, _NMAIN, _CH)
    dst = dst_f[:, : _NMAIN * _CH].reshape(_NS, _NMAIN, _CH)
    src_t = src_f[:, _NMAIN * _CH :].reshape(_NS, 1, _TAIL)
    dst_t = dst_f[:, _NMAIN * _CH :].reshape(_NS, 1, _TAIL)
    x_split = jnp.stack([x[:, : _D // 2], x[:, _D // 2 :]])
    zrow = jnp.zeros((_NPAD, _D // 2), jnp.float32)
    zdeg = jnp.zeros((_NPAD, _DEGW), jnp.float32)
    zout = jnp.zeros((_NPAD, _O // 2), jnp.float32)
    ones = jnp.ones((_CH, _DEGW), jnp.float32)

    agg1, degp = _sc_agg_deg(x_split, src, dst, src_t, dst_t, zrow, zdeg, ones)
    y2s, r2 = _tc1(agg1[:, : _N], degp[: _N], x, W1_l, b1.reshape(1, _D),
                   W1_r, W2_l, W2_r)
    (agg2,) = _sc_agg_only(y2s, src, dst, src_t, dst_t, zout)
    return _tc2(agg2[:, : _N], degp[: _N], r2, b2.reshape(1, _O))


# speedup vs baseline: 11.0612x; 1.0628x over previous
import functools

import jax
import jax.numpy as jnp
from jax import lax
from jax.experimental import pallas as pl
from jax.experimental.pallas import tpu as pltpu
from jax.experimental.pallas import tpu_sc as plsc

_N = 10000
_E = 320000
_D = 128
_O = 64

_NC = 2
_NS = 16
_CH = 128
_EPT = _E // _NS
_NMAIN = _EPT // _CH
_TAIL = _EPT - _NMAIN * _CH
_NPAD = 10112
_RPT = _NPAD // _NS
_DEGW = 8


def _sc_agg_body(with_deg, *refs):
    if with_deg:
        (y_hbm, src_hbm, dst_hbm, srct_hbm, dstt_hbm, zrow_hbm, zdeg_hbm,
         ones_hbm, agg_out, deg_out, srcv, dstv, srctv, dsttv, rows, rowst,
         onesv, agg_s, deg_s, sem0, sem1) = refs
    else:
        (y_hbm, src_hbm, dst_hbm, srct_hbm, dstt_hbm, zrow_hbm, agg_out,
         srcv, dstv, srctv, dsttv, rows, rowst, agg_s, sem0, sem1) = refs
    sems = (sem0, sem1)

    cid = lax.axis_index("c")
    sid = lax.axis_index("s")

    r0 = sid * _RPT
    pltpu.sync_copy(zrow_hbm.at[pl.ds(r0, _RPT)], agg_s.at[pl.ds(r0, _RPT)])
    if with_deg:
        pltpu.sync_copy(zdeg_hbm.at[pl.ds(r0, _RPT)], deg_s.at[pl.ds(r0, _RPT)])
        pltpu.sync_copy(ones_hbm, onesv)

    pltpu.sync_copy(src_hbm.at[sid], srcv)
    pltpu.sync_copy(dst_hbm.at[sid], dstv)
    pltpu.sync_copy(srct_hbm.at[sid], srctv)
    pltpu.sync_copy(dstt_hbm.at[sid], dsttv)
    plsc.subcore_barrier()

    pltpu.async_copy(y_hbm.at[cid].at[srcv.at[0]], rows.at[0], sems[0])
    pltpu.async_copy(y_hbm.at[cid].at[srcv.at[1]], rows.at[1], sems[1])

    def pair(i, _):
        for b in range(2):
            c = 2 * i + b
            pltpu.make_async_copy(
                y_hbm.at[cid].at[srcv.at[c]], rows.at[b], sems[b]).wait()
            pltpu.sync_copy(rows.at[b], agg_s.at[dstv.at[c]], add=True)
            if with_deg:
                @pl.when(cid == 0)
                def _():
                    pltpu.sync_copy(onesv, deg_s.at[dstv.at[c]], add=True)

            @pl.when(c + 2 < _NMAIN)
            def _():
                pltpu.async_copy(
                    y_hbm.at[cid].at[srcv.at[c + 2]], rows.at[b], sems[b])
        return 0

    lax.fori_loop(0, _NMAIN // 2, pair, 0)

    pltpu.async_copy(y_hbm.at[cid].at[srctv.at[0]], rowst, sems[0]).wait()
    pltpu.sync_copy(rowst, agg_s.at[dsttv.at[0]], add=True)
    if with_deg:
        @pl.when(cid == 0)
        def _():
            pltpu.sync_copy(onesv.at[pl.ds(0, _TAIL)],
                            deg_s.at[dsttv.at[0]], add=True)

    plsc.subcore_barrier()
    pltpu.sync_copy(agg_s.at[pl.ds(r0, _RPT)], agg_out.at[cid, pl.ds(r0, _RPT)])
    if with_deg:
        @pl.when(cid == 0)
        def _():
            pltpu.sync_copy(deg_s.at[pl.ds(r0, _RPT)], deg_out.at[pl.ds(r0, _RPT)])


def _make_sc_agg(d, with_deg):
    mesh = plsc.VectorSubcoreMesh(core_axis_name="c", subcore_axis_name="s")
    out_type = [jax.ShapeDtypeStruct((_NC, _NPAD, d), jnp.float32)]
    scratch = [
        pltpu.VMEM((_NMAIN, _CH), jnp.int32),
        pltpu.VMEM((_NMAIN, _CH), jnp.int32),
        pltpu.VMEM((1, _TAIL), jnp.int32),
        pltpu.VMEM((1, _TAIL), jnp.int32),
        pltpu.VMEM((2, _CH, d), jnp.float32),
        pltpu.VMEM((_TAIL, d), jnp.float32),
    ]
    if with_deg:
        out_type.append(jax.ShapeDtypeStruct((_NPAD, _DEGW), jnp.float32))
        scratch.append(pltpu.VMEM((_CH, _DEGW), jnp.float32))
        scratch.append(pltpu.VMEM_SHARED((_NPAD, d), jnp.float32))
        scratch.append(pltpu.VMEM_SHARED((_NPAD, _DEGW), jnp.float32))
    else:
        scratch.append(pltpu.VMEM_SHARED((_NPAD, d), jnp.float32))
    scratch.append(pltpu.SemaphoreType.DMA)
    scratch.append(pltpu.SemaphoreType.DMA)
    return pl.kernel(
        functools.partial(_sc_agg_body, with_deg),
        out_type=out_type,
        mesh=mesh,
        scratch_types=scratch,
        compiler_params=pltpu.CompilerParams(use_tc_tiling_on_sc=False),
    )


_sc_agg_deg = _make_sc_agg(_D // 2, True)
_sc_agg_only = _make_sc_agg(_O // 2, False)


def _tc1_body(agg_ref, deg_ref, x_ref, w1l_ref, b1_ref, w1r_ref,
              w2l_ref, w2r_ref, y2_ref, r2_ref):
    agg = jnp.concatenate([agg_ref[0], agg_ref[1]], axis=1)
    deg = deg_ref[:, 0:1]
    mean = agg * (1.0 / jnp.maximum(deg, 1.0))
    h = mean @ w1l_ref[...] + b1_ref[...] + x_ref[...] @ w1r_ref[...]
    h = jnp.maximum(h, 0.0)
    y2_ref[0] = h @ w2l_ref[:, : _O // 2]
    y2_ref[1] = h @ w2l_ref[:, _O // 2 :]
    r2_ref[...] = h @ w2r_ref[...]


def _tc2_body(agg_ref, deg_ref, r2_ref, b2_ref, z_ref):
    agg = jnp.concatenate([agg_ref[0], agg_ref[1]], axis=1)
    deg = deg_ref[:, 0:1]
    z_ref[...] = agg * (1.0 / jnp.maximum(deg, 1.0)) + b2_ref[...] + r2_ref[...]


_TC_R = 1000


def _tc1(agg, degp, x, w1l, b1, w1r, w2l, w2r):
    nb = _N // _TC_R
    return pl.pallas_call(
        _tc1_body,
        grid=(nb,),
        in_specs=[
            pl.BlockSpec((_NC, _TC_R, _D // 2), lambda m: (0, m, 0)),
            pl.BlockSpec((_TC_R, _DEGW), lambda m: (m, 0)),
            pl.BlockSpec((_TC_R, _D), lambda m: (m, 0)),
            pl.BlockSpec((_D, _D), lambda m: (0, 0)),
            pl.BlockSpec((1, _D), lambda m: (0, 0)),
            pl.BlockSpec((_D, _D), lambda m: (0, 0)),
            pl.BlockSpec((_D, _O), lambda m: (0, 0)),
            pl.BlockSpec((_D, _O), lambda m: (0, 0)),
        ],
        out_specs=[
            pl.BlockSpec((_NC, _TC_R, _O // 2), lambda m: (0, m, 0)),
            pl.BlockSpec((_TC_R, _O), lambda m: (m, 0)),
        ],
        out_shape=[
            jax.ShapeDtypeStruct((_NC, _N, _O // 2), jnp.float32),
            jax.ShapeDtypeStruct((_N, _O), jnp.float32),
        ],
    )(agg, degp, x, w1l, b1, w1r, w2l, w2r)


def _tc2(agg2, degp, r2, b2):
    nb = _N // _TC_R
    return pl.pallas_call(
        _tc2_body,
        grid=(nb,),
        in_specs=[
            pl.BlockSpec((_NC, _TC_R, _O // 2), lambda m: (0, m, 0)),
            pl.BlockSpec((_TC_R, _DEGW), lambda m: (m, 0)),
            pl.BlockSpec((_TC_R, _O), lambda m: (m, 0)),
            pl.BlockSpec((1, _O), lambda m: (0, 0)),
        ],
        out_specs=pl.BlockSpec((_TC_R, _O), lambda m: (m, 0)),
        out_shape=jax.ShapeDtypeStruct((_N, _O), jnp.float32),
    )(agg2, degp, r2, b2)


def kernel(x, edge_index, W1_l, b1, W1_r, W2_l, b2, W2_r):
    nm = _NS * _NMAIN * _CH
    src = lax.slice(edge_index[0], (0,), (nm,)).reshape(_NS, _NMAIN, _CH)
    dst = lax.slice(edge_index[1], (0,), (nm,)).reshape(_NS, _NMAIN, _CH)
    src_t = lax.slice(edge_index[0], (nm,), (_E,)).reshape(_NS, 1, _TAIL)
    dst_t = lax.slice(edge_index[1], (nm,), (_E,)).reshape(_NS, 1, _TAIL)
    x_split = jnp.stack([x[:, : _D // 2], x[:, _D // 2 :]])
    zrow = jnp.zeros((_NPAD, _D // 2), jnp.float32)
    zdeg = jnp.zeros((_NPAD, _DEGW), jnp.float32)
    zout = jnp.zeros((_NPAD, _O // 2), jnp.float32)
    ones = jnp.ones((_CH, _DEGW), jnp.float32)

    agg1, degp = _sc_agg_deg(x_split, src, dst, src_t, dst_t, zrow, zdeg, ones)
    y2s, r2 = _tc1(agg1, degp, x, W1_l, b1.reshape(1, _D), W1_r, W2_l, W2_r)
    (agg2,) = _sc_agg_only(y2s, src, dst, src_t, dst_t, zout)
    return _tc2(agg2, degp, r2, b2.reshape(1, _O))


# speedup vs baseline: 13.5292x; 1.2231x over previous
import functools

import jax
import jax.numpy as jnp
from jax import lax
from jax.experimental import pallas as pl
from jax.experimental.pallas import tpu as pltpu
from jax.experimental.pallas import tpu_sc as plsc

_N = 10000
_E = 320000
_D = 128
_O = 64

_NC = 2
_NS = 16
_CH = 128
_EPT = _E // _NS
_NMAIN = _EPT // _CH
_TAIL = _EPT - _NMAIN * _CH
_NPAD = 10112
_RPT = _NPAD // _NS
_DEGW = 8
_NBUF = 4


def _sc_agg_body(with_deg, *refs):
    if with_deg:
        (y_hbm, src_hbm, dst_hbm, srct_hbm, dstt_hbm, zrow_hbm, zdeg_hbm,
         ones_hbm, agg_out, deg_out, srcv, dstv, srctv, dsttv, rows, rowst,
         onesv, agg_s, deg_s, sem0, sem1, sem2, sem3) = refs
    else:
        (y_hbm, src_hbm, dst_hbm, srct_hbm, dstt_hbm, zrow_hbm, agg_out,
         srcv, dstv, srctv, dsttv, rows, rowst, agg_s, sem0, sem1, sem2,
         sem3) = refs
    sems = (sem0, sem1, sem2, sem3)

    cid = lax.axis_index("c")
    sid = lax.axis_index("s")

    r0 = sid * _RPT
    pltpu.sync_copy(zrow_hbm.at[pl.ds(r0, _RPT)], agg_s.at[pl.ds(r0, _RPT)])
    if with_deg:
        pltpu.sync_copy(zdeg_hbm.at[pl.ds(r0, _RPT)], deg_s.at[pl.ds(r0, _RPT)])
        pltpu.sync_copy(ones_hbm, onesv)

    pltpu.sync_copy(src_hbm.at[sid], srcv)
    pltpu.sync_copy(dst_hbm.at[sid], dstv)
    pltpu.sync_copy(srct_hbm.at[sid], srctv)
    pltpu.sync_copy(dstt_hbm.at[sid], dsttv)
    plsc.subcore_barrier()

    for b in range(_NBUF):
        pltpu.async_copy(y_hbm.at[cid].at[srcv.at[b]], rows.at[b], sems[b])

    def ring(i, _):
        for b in range(_NBUF):
            c = _NBUF * i + b
            pltpu.make_async_copy(
                y_hbm.at[cid].at[srcv.at[c]], rows.at[b], sems[b]).wait()
            pltpu.sync_copy(rows.at[b], agg_s.at[dstv.at[c]], add=True)
            if with_deg:
                @pl.when((cid == 0) == (c < _NMAIN // 2))
                def _():
                    pltpu.sync_copy(onesv, deg_s.at[dstv.at[c]], add=True)

            @pl.when(c + _NBUF < _NMAIN)
            def _():
                pltpu.async_copy(
                    y_hbm.at[cid].at[srcv.at[c + _NBUF]], rows.at[b], sems[b])
        return 0

    lax.fori_loop(0, _NMAIN // _NBUF, ring, 0)

    pltpu.async_copy(y_hbm.at[cid].at[srctv.at[0]], rowst, sems[0]).wait()
    pltpu.sync_copy(rowst, agg_s.at[dsttv.at[0]], add=True)
    if with_deg:
        @pl.when(cid == 1)
        def _():
            pltpu.sync_copy(onesv.at[pl.ds(0, _TAIL)],
                            deg_s.at[dsttv.at[0]], add=True)

    plsc.subcore_barrier()
    pltpu.sync_copy(agg_s.at[pl.ds(r0, _RPT)], agg_out.at[cid, pl.ds(r0, _RPT)])
    if with_deg:
        pltpu.sync_copy(deg_s.at[pl.ds(r0, _RPT)],
                        deg_out.at[cid, pl.ds(r0, _RPT)])


def _make_sc_agg(d, with_deg):
    mesh = plsc.VectorSubcoreMesh(core_axis_name="c", subcore_axis_name="s")
    out_type = [jax.ShapeDtypeStruct((_NC, _NPAD, d), jnp.float32)]
    scratch = [
        pltpu.VMEM((_NMAIN, _CH), jnp.int32),
        pltpu.VMEM((_NMAIN, _CH), jnp.int32),
        pltpu.VMEM((1, _TAIL), jnp.int32),
        pltpu.VMEM((1, _TAIL), jnp.int32),
        pltpu.VMEM((_NBUF, _CH, d), jnp.float32),
        pltpu.VMEM((_TAIL, d), jnp.float32),
    ]
    if with_deg:
        out_type.append(jax.ShapeDtypeStruct((_NC, _NPAD, _DEGW), jnp.float32))
        scratch.append(pltpu.VMEM((_CH, _DEGW), jnp.float32))
        scratch.append(pltpu.VMEM_SHARED((_NPAD, d), jnp.float32))
        scratch.append(pltpu.VMEM_SHARED((_NPAD, _DEGW), jnp.float32))
    else:
        scratch.append(pltpu.VMEM_SHARED((_NPAD, d), jnp.float32))
    for _ in range(_NBUF):
        scratch.append(pltpu.SemaphoreType.DMA)
    return pl.kernel(
        functools.partial(_sc_agg_body, with_deg),
        out_type=out_type,
        mesh=mesh,
        scratch_types=scratch,
        compiler_params=pltpu.CompilerParams(use_tc_tiling_on_sc=False),
    )


_sc_agg_deg = _make_sc_agg(_D // 2, True)
_sc_agg_only = _make_sc_agg(_O // 2, False)


def _tc1_body(agg_ref, deg_ref, x_ref, w1l_ref, b1_ref, w1r_ref,
              w2l_ref, w2r_ref, y2_ref, r2_ref):
    agg = jnp.concatenate([agg_ref[0], agg_ref[1]], axis=1)
    deg = deg_ref[0, :, 0:1] + deg_ref[1, :, 0:1]
    mean = agg * (1.0 / jnp.maximum(deg, 1.0))
    h = mean @ w1l_ref[...] + b1_ref[...] + x_ref[...] @ w1r_ref[...]
    h = jnp.maximum(h, 0.0)
    y2_ref[0] = h @ w2l_ref[:, : _O // 2]
    y2_ref[1] = h @ w2l_ref[:, _O // 2 :]
    r2_ref[...] = h @ w2r_ref[...]


def _tc2_body(agg_ref, deg_ref, r2_ref, b2_ref, z_ref):
    agg = jnp.concatenate([agg_ref[0], agg_ref[1]], axis=1)
    deg = deg_ref[0, :, 0:1] + deg_ref[1, :, 0:1]
    z_ref[...] = agg * (1.0 / jnp.maximum(deg, 1.0)) + b2_ref[...] + r2_ref[...]


_TC_R = 1000


def _tc1(agg, degp, x, w1l, b1, w1r, w2l, w2r):
    nb = _N // _TC_R
    return pl.pallas_call(
        _tc1_body,
        grid=(nb,),
        in_specs=[
            pl.BlockSpec((_NC, _TC_R, _D // 2), lambda m: (0, m, 0)),
            pl.BlockSpec((_NC, _TC_R, _DEGW), lambda m: (0, m, 0)),
            pl.BlockSpec((_TC_R, _D), lambda m: (m, 0)),
            pl.BlockSpec((_D, _D), lambda m: (0, 0)),
            pl.BlockSpec((1, _D), lambda m: (0, 0)),
            pl.BlockSpec((_D, _D), lambda m: (0, 0)),
            pl.BlockSpec((_D, _O), lambda m: (0, 0)),
            pl.BlockSpec((_D, _O), lambda m: (0, 0)),
        ],
        out_specs=[
            pl.BlockSpec((_NC, _TC_R, _O // 2), lambda m: (0, m, 0)),
            pl.BlockSpec((_TC_R, _O), lambda m: (m, 0)),
        ],
        out_shape=[
            jax.ShapeDtypeStruct((_NC, _N, _O // 2), jnp.float32),
            jax.ShapeDtypeStruct((_N, _O), jnp.float32),
        ],
    )(agg, degp, x, w1l, b1, w1r, w2l, w2r)


def _tc2(agg2, degp, r2, b2):
    nb = _N // _TC_R
    return pl.pallas_call(
        _tc2_body,
        grid=(nb,),
        in_specs=[
            pl.BlockSpec((_NC, _TC_R, _O // 2), lambda m: (0, m, 0)),
            pl.BlockSpec((_NC, _TC_R, _DEGW), lambda m: (0, m, 0)),
            pl.BlockSpec((_TC_R, _O), lambda m: (m, 0)),
            pl.BlockSpec((1, _O), lambda m: (0, 0)),
        ],
        out_specs=pl.BlockSpec((_TC_R, _O), lambda m: (m, 0)),
        out_shape=jax.ShapeDtypeStruct((_N, _O), jnp.float32),
    )(agg2, degp, r2, b2)


def kernel(x, edge_index, W1_l, b1, W1_r, W2_l, b2, W2_r):
    nm = _NS * _NMAIN * _CH
    src = lax.slice(edge_index[0], (0,), (nm,)).reshape(_NS, _NMAIN, _CH)
    dst = lax.slice(edge_index[1], (0,), (nm,)).reshape(_NS, _NMAIN, _CH)
    src_t = lax.slice(edge_index[0], (nm,), (_E,)).reshape(_NS, 1, _TAIL)
    dst_t = lax.slice(edge_index[1], (nm,), (_E,)).reshape(_NS, 1, _TAIL)
    x_split = jnp.stack([x[:, : _D // 2], x[:, _D // 2 :]])
    zrow = jnp.zeros((_NPAD, _D // 2), jnp.float32)
    zdeg = jnp.zeros((_NPAD, _DEGW), jnp.float32)
    zout = jnp.zeros((_NPAD, _O // 2), jnp.float32)
    ones = jnp.ones((_CH, _DEGW), jnp.float32)

    agg1, degp = _sc_agg_deg(x_split, src, dst, src_t, dst_t, zrow, zdeg, ones)
    y2s, r2 = _tc1(agg1, degp, x, W1_l, b1.reshape(1, _D), W1_r, W2_l, W2_r)
    (agg2,) = _sc_agg_only(y2s, src, dst, src_t, dst_t, zout)
    return _tc2(agg2, degp, r2, b2.reshape(1, _O))


# speedup vs baseline: 14.8681x; 1.0990x over previous
import functools

import jax
import jax.numpy as jnp
from jax import lax
from jax.experimental import pallas as pl
from jax.experimental.pallas import tpu as pltpu
from jax.experimental.pallas import tpu_sc as plsc

_N = 10000
_E = 320000
_D = 128
_O = 64

_NC = 2
_NS = 16
_CH = 128
_EPT = _E // _NS
_NMAIN = _EPT // _CH
_TAIL = _EPT - _NMAIN * _CH
_NPAD = 10112
_RPT = _NPAD // _NS
_DEGW = 8
_NBUF = 4


def _sc_agg_body(with_deg, *refs):
    if with_deg:
        (y_hbm, edge_hbm, zrow_hbm, zdeg_hbm, ones_hbm, agg_out, deg_out,
         srcv, dstv, rows, rowst, onesv, agg_s, deg_s,
         sem0, sem1, sem2, sem3) = refs
    else:
        (y_hbm, edge_hbm, zrow_hbm, agg_out,
         srcv, dstv, rows, rowst, agg_s, sem0, sem1, sem2, sem3) = refs
    sems = (sem0, sem1, sem2, sem3)

    cid = lax.axis_index("c")
    sid = lax.axis_index("s")

    r0 = sid * _RPT
    pltpu.sync_copy(zrow_hbm.at[pl.ds(r0, _RPT)], agg_s.at[pl.ds(r0, _RPT)])
    if with_deg:
        pltpu.sync_copy(zdeg_hbm.at[pl.ds(r0, _RPT)], deg_s.at[pl.ds(r0, _RPT)])
        pltpu.sync_copy(ones_hbm, onesv)

    e0 = sid * _EPT
    pltpu.sync_copy(edge_hbm.at[0, pl.ds(e0, _EPT)], srcv)
    pltpu.sync_copy(edge_hbm.at[1, pl.ds(e0, _EPT)], dstv)
    plsc.subcore_barrier()

    for b in range(_NBUF):
        pltpu.async_copy(y_hbm.at[cid].at[srcv.at[pl.ds(b * _CH, _CH)]],
                         rows.at[b], sems[b])

    def ring(i, _):
        for b in range(_NBUF):
            c = _NBUF * i + b
            pltpu.make_async_copy(
                y_hbm.at[cid].at[srcv.at[pl.ds(c * _CH, _CH)]],
                rows.at[b], sems[b]).wait()
            pltpu.sync_copy(
                rows.at[b], agg_s.at[dstv.at[pl.ds(c * _CH, _CH)]], add=True)
            if with_deg:
                @pl.when((cid == 0) == (c < _NMAIN // 2))
                def _():
                    pltpu.sync_copy(
                        onesv, deg_s.at[dstv.at[pl.ds(c * _CH, _CH)]], add=True)

            @pl.when(c + _NBUF < _NMAIN)
            def _():
                pltpu.async_copy(
                    y_hbm.at[cid].at[srcv.at[pl.ds((c + _NBUF) * _CH, _CH)]],
                    rows.at[b], sems[b])
        return 0

    lax.fori_loop(0, _NMAIN // _NBUF, ring, 0)

    t0 = _NMAIN * _CH
    pltpu.async_copy(y_hbm.at[cid].at[srcv.at[pl.ds(t0, _TAIL)]],
                     rowst, sems[0]).wait()
    pltpu.sync_copy(rowst, agg_s.at[dstv.at[pl.ds(t0, _TAIL)]], add=True)
    if with_deg:
        @pl.when(cid == 1)
        def _():
            pltpu.sync_copy(onesv.at[pl.ds(0, _TAIL)],
                            deg_s.at[dstv.at[pl.ds(t0, _TAIL)]], add=True)

    plsc.subcore_barrier()
    pltpu.sync_copy(agg_s.at[pl.ds(r0, _RPT)], agg_out.at[cid, pl.ds(r0, _RPT)])
    if with_deg:
        pltpu.sync_copy(deg_s.at[pl.ds(r0, _RPT)],
                        deg_out.at[cid, pl.ds(r0, _RPT)])


def _make_sc_agg(d, with_deg):
    mesh = plsc.VectorSubcoreMesh(core_axis_name="c", subcore_axis_name="s")
    out_type = [jax.ShapeDtypeStruct((_NC, _NPAD, d), jnp.float32)]
    scratch = [
        pltpu.VMEM((_EPT,), jnp.int32),
        pltpu.VMEM((_EPT,), jnp.int32),
        pltpu.VMEM((_NBUF, _CH, d), jnp.float32),
        pltpu.VMEM((_TAIL, d), jnp.float32),
    ]
    if with_deg:
        out_type.append(jax.ShapeDtypeStruct((_NC, _NPAD, _DEGW), jnp.float32))
        scratch.append(pltpu.VMEM((_CH, _DEGW), jnp.float32))
        scratch.append(pltpu.VMEM_SHARED((_NPAD, d), jnp.float32))
        scratch.append(pltpu.VMEM_SHARED((_NPAD, _DEGW), jnp.float32))
    else:
        scratch.append(pltpu.VMEM_SHARED((_NPAD, d), jnp.float32))
    for _ in range(_NBUF):
        scratch.append(pltpu.SemaphoreType.DMA)
    return pl.kernel(
        functools.partial(_sc_agg_body, with_deg),
        out_type=out_type,
        mesh=mesh,
        scratch_types=scratch,
        compiler_params=pltpu.CompilerParams(use_tc_tiling_on_sc=False),
    )


_sc_agg_deg = _make_sc_agg(_D // 2, True)
_sc_agg_only = _make_sc_agg(_O // 2, False)


def _tc1_body(agg_ref, deg_ref, x_ref, w1l_ref, b1_ref, w1r_ref,
              w2l_ref, w2r_ref, y2_ref, r2_ref):
    agg = jnp.concatenate([agg_ref[0], agg_ref[1]], axis=1)
    deg = deg_ref[0, :, 0:1] + deg_ref[1, :, 0:1]
    mean = agg * (1.0 / jnp.maximum(deg, 1.0))
    h = mean @ w1l_ref[...] + b1_ref[...] + x_ref[...] @ w1r_ref[...]
    h = jnp.maximum(h, 0.0)
    y2_ref[0] = h @ w2l_ref[:, : _O // 2]
    y2_ref[1] = h @ w2l_ref[:, _O // 2 :]
    r2_ref[...] = h @ w2r_ref[...]


def _tc2_body(agg_ref, deg_ref, r2_ref, b2_ref, z_ref):
    agg = jnp.concatenate([agg_ref[0], agg_ref[1]], axis=1)
    deg = deg_ref[0, :, 0:1] + deg_ref[1, :, 0:1]
    z_ref[...] = agg * (1.0 / jnp.maximum(deg, 1.0)) + b2_ref[...] + r2_ref[...]


_TC_R = 1000


def _tc1(agg, degp, x, w1l, b1, w1r, w2l, w2r):
    nb = _N // _TC_R
    return pl.pallas_call(
        _tc1_body,
        grid=(nb,),
        in_specs=[
            pl.BlockSpec((_NC, _TC_R, _D // 2), lambda m: (0, m, 0)),
            pl.BlockSpec((_NC, _TC_R, _DEGW), lambda m: (0, m, 0)),
            pl.BlockSpec((_TC_R, _D), lambda m: (m, 0)),
            pl.BlockSpec((_D, _D), lambda m: (0, 0)),
            pl.BlockSpec((1, _D), lambda m: (0, 0)),
            pl.BlockSpec((_D, _D), lambda m: (0, 0)),
            pl.BlockSpec((_D, _O), lambda m: (0, 0)),
            pl.BlockSpec((_D, _O), lambda m: (0, 0)),
        ],
        out_specs=[
            pl.BlockSpec((_NC, _TC_R, _O // 2), lambda m: (0, m, 0)),
            pl.BlockSpec((_TC_R, _O), lambda m: (m, 0)),
        ],
        out_shape=[
            jax.ShapeDtypeStruct((_NC, _N, _O // 2), jnp.float32),
            jax.ShapeDtypeStruct((_N, _O), jnp.float32),
        ],
    )(agg, degp, x, w1l, b1, w1r, w2l, w2r)


def _tc2(agg2, degp, r2, b2):
    nb = _N // _TC_R
    return pl.pallas_call(
        _tc2_body,
        grid=(nb,),
        in_specs=[
            pl.BlockSpec((_NC, _TC_R, _O // 2), lambda m: (0, m, 0)),
            pl.BlockSpec((_NC, _TC_R, _DEGW), lambda m: (0, m, 0)),
            pl.BlockSpec((_TC_R, _O), lambda m: (m, 0)),
            pl.BlockSpec((1, _O), lambda m: (0, 0)),
        ],
        out_specs=pl.BlockSpec((_TC_R, _O), lambda m: (m, 0)),
        out_shape=jax.ShapeDtypeStruct((_N, _O), jnp.float32),
    )(agg2, degp, r2, b2)


def kernel(x, edge_index, W1_l, b1, W1_r, W2_l, b2, W2_r):
    x_split = jnp.stack([x[:, : _D // 2], x[:, _D // 2 :]])
    zrow = jnp.zeros((_NPAD, _D // 2), jnp.float32)
    zdeg = jnp.zeros((_NPAD, _DEGW), jnp.float32)
    zout = jnp.zeros((_NPAD, _O // 2), jnp.float32)
    ones = jnp.ones((_CH, _DEGW), jnp.float32)

    agg1, degp = _sc_agg_deg(x_split, edge_index, zrow, zdeg, ones)
    y2s, r2 = _tc1(agg1, degp, x, W1_l, b1.reshape(1, _D), W1_r, W2_l, W2_r)
    (agg2,) = _sc_agg_only(y2s, edge_index, zout)
    return _tc2(agg2, degp, r2, b2.reshape(1, _O))
